# baseline (speedup 1.0000x reference)
import jax
import jax.numpy as jnp
from jax.experimental import pallas as pl
from jax.experimental.pallas import tpu as pltpu


def _pick_row_block(n, pref):
    for cand in pref:
        if n % cand == 0:
            return cand
    return n


def _dinv(deg):
    return jnp.where(deg > 0, jax.lax.rsqrt(deg), 0.0)


def _pass1_kernel(adj_ref, x_ref, w1_ref, deg_ref, adjb_ref, z_ref):
    m = adj_ref.shape[0]
    a = adj_ref[...]
    deg_ref[...] = (jnp.sum(a, axis=1) + 1.0).reshape(m, 1)
    adjb_ref[...] = a.astype(jnp.bfloat16)
    z_ref[...] = jnp.dot(x_ref[...], w1_ref[...], preferred_element_type=jnp.float32)


def _layers_kernel(adj_ref, z_ref, degf_ref, deg_ref, b1_ref, w2_ref, b2_ref,
                   out_ref, y1_scr, y2_scr):
    p = pl.program_id(0)
    i = pl.program_id(1)
    m = adj_ref.shape[0]

    @pl.when((p == 0) & (i == 0))
    def _():
        y1_scr[...] = (_dinv(degf_ref[...]) * z_ref[...]).astype(jnp.bfloat16)

    @pl.when(p == 0)
    def _():
        dinv = _dinv(deg_ref[...])
        acc = jnp.dot(adj_ref[...], y1_scr[...], preferred_element_type=jnp.float32)
        yself = y1_scr[pl.ds(i * m, m), :].astype(jnp.float32)
        x1 = dinv * (acc + yself) + b1_ref[...]
        x1 = jnp.maximum(x1, 0.0)
        y2_scr[pl.ds(i * m, m), :] = (
            dinv * jnp.dot(x1, w2_ref[...], preferred_element_type=jnp.float32)
        ).astype(jnp.bfloat16)

    @pl.when(p == 1)
    def _():
        dinv = _dinv(deg_ref[...])
        acc = jnp.dot(adj_ref[...], y2_scr[...], preferred_element_type=jnp.float32)
        yself = y2_scr[pl.ds(i * m, m), :].astype(jnp.float32)
        out_ref[...] = dinv * (acc + yself) + b2_ref[...]


@jax.jit
def kernel(feature, adj, W1, b1, W2, b2):
    n, d = feature.shape
    h1 = W1.shape[1]
    h2 = W2.shape[1]
    mb = _pick_row_block(n, (400, 200, 80, 40, 16, 8))
    nmb = n // mb
    lb = _pick_row_block(n, (400, 200, 80, 40, 16, 8))
    nlb = n // lb

    deg, adjb, z = pl.pallas_call(
        _pass1_kernel,
        grid=(nmb,),
        in_specs=[
            pl.BlockSpec((mb, n), lambda i: (i, 0)),
            pl.BlockSpec((mb, d), lambda i: (i, 0)),
            pl.BlockSpec((d, h1), lambda i: (0, 0)),
        ],
        out_specs=[
            pl.BlockSpec((mb, 1), lambda i: (i, 0)),
            pl.BlockSpec((mb, n), lambda i: (i, 0)),
            pl.BlockSpec((mb, h1), lambda i: (i, 0)),
        ],
        out_shape=[
            jax.ShapeDtypeStruct((n, 1), jnp.float32),
            jax.ShapeDtypeStruct((n, n), jnp.bfloat16),
            jax.ShapeDtypeStruct((n, h1), jnp.float32),
        ],
    )(adj, feature, W1)

    b1r = b1.reshape(1, h1)
    b2r = b2.reshape(1, h2)

    x2 = pl.pallas_call(
        _layers_kernel,
        grid=(2, nlb),
        in_specs=[
            pl.BlockSpec((lb, n), lambda p, i: (i, 0)),
            pl.BlockSpec((n, h1), lambda p, i: (0, 0)),
            pl.BlockSpec((n, 1), lambda p, i: (0, 0)),
            pl.BlockSpec((lb, 1), lambda p, i: (i, 0)),
            pl.BlockSpec((1, h1), lambda p, i: (0, 0)),
            pl.BlockSpec((h1, h2), lambda p, i: (0, 0)),
            pl.BlockSpec((1, h2), lambda p, i: (0, 0)),
        ],
        out_specs=pl.BlockSpec((lb, h2), lambda p, i: (i, 0)),
        out_shape=jax.ShapeDtypeStruct((n, h2), jnp.float32),
        scratch_shapes=[
            pltpu.VMEM((n, h1), jnp.bfloat16),
            pltpu.VMEM((n, h2), jnp.bfloat16),
        ],
    )(adjb, z, deg, deg, b1r, W2, b2r)

    return x2


# speedup vs baseline: 1.0540x; 1.0540x over previous
import jax
import jax.numpy as jnp
from jax.experimental import pallas as pl
from jax.experimental.pallas import tpu as pltpu


def _pick_row_block(n, pref):
    for cand in pref:
        if n % cand == 0:
            return cand
    return n


def _dinv(deg):
    return jnp.where(deg > 0, jax.lax.rsqrt(deg), 0.0)


def _pass1_kernel(adj_ref, x_ref, w1_ref, deg_ref, adjb_ref, z_ref):
    m = adj_ref.shape[0]
    a = adj_ref[...]
    deg_ref[...] = (jnp.sum(a, axis=1) + 1.0).reshape(m, 1)
    adjb_ref[...] = a.astype(jnp.bfloat16)
    z_ref[...] = jnp.dot(x_ref[...], w1_ref[...], preferred_element_type=jnp.float32)


def _layers_kernel(adj_ref, z_ref, degf_ref, deg_ref, b1_ref, w2_ref, b2_ref,
                   out_ref, y1_scr, y2_scr):
    p = pl.program_id(0)
    i = pl.program_id(1)
    m = adj_ref.shape[0]

    @pl.when((p == 0) & (i == 0))
    def _():
        y1_scr[...] = (_dinv(degf_ref[...]) * z_ref[...]).astype(jnp.bfloat16)

    @pl.when(p == 0)
    def _():
        dinv = _dinv(deg_ref[...])
        acc = jnp.dot(adj_ref[...], y1_scr[...], preferred_element_type=jnp.float32)
        yself = y1_scr[pl.ds(i * m, m), :].astype(jnp.float32)
        x1 = dinv * (acc + yself) + b1_ref[...]
        x1 = jnp.maximum(x1, 0.0)
        y2_scr[pl.ds(i * m, m), :] = (
            dinv * jnp.dot(x1, w2_ref[...], preferred_element_type=jnp.float32)
        ).astype(jnp.bfloat16)

    @pl.when(p == 1)
    def _():
        dinv = _dinv(deg_ref[...])
        acc = jnp.dot(adj_ref[...], y2_scr[...], preferred_element_type=jnp.float32)
        yself = y2_scr[pl.ds(i * m, m), :].astype(jnp.float32)
        out_ref[...] = dinv * (acc + yself) + b2_ref[...]


@jax.jit
def kernel(feature, adj, W1, b1, W2, b2):
    n, d = feature.shape
    h1 = W1.shape[1]
    h2 = W2.shape[1]
    mb = _pick_row_block(n, (400, 200, 80, 40, 16, 8))
    nmb = n // mb
    lb = _pick_row_block(n, (1000, 400, 200, 80, 40, 16, 8))
    nlb = n // lb

    deg, adjb, z = pl.pallas_call(
        _pass1_kernel,
        grid=(nmb,),
        in_specs=[
            pl.BlockSpec((mb, n), lambda i: (i, 0)),
            pl.BlockSpec((mb, d), lambda i: (i, 0)),
            pl.BlockSpec((d, h1), lambda i: (0, 0)),
        ],
        out_specs=[
            pl.BlockSpec((mb, 1), lambda i: (i, 0)),
            pl.BlockSpec((mb, n), lambda i: (i, 0)),
            pl.BlockSpec((mb, h1), lambda i: (i, 0)),
        ],
        out_shape=[
            jax.ShapeDtypeStruct((n, 1), jnp.float32),
            jax.ShapeDtypeStruct((n, n), jnp.bfloat16),
            jax.ShapeDtypeStruct((n, h1), jnp.float32),
        ],
    )(adj, feature, W1)

    b1r = b1.reshape(1, h1)
    b2r = b2.reshape(1, h2)

    x2 = pl.pallas_call(
        _layers_kernel,
        grid=(2, nlb),
        in_specs=[
            pl.BlockSpec((lb, n), lambda p, i: (i, 0)),
            pl.BlockSpec((n, h1), lambda p, i: (0, 0)),
            pl.BlockSpec((n, 1), lambda p, i: (0, 0)),
            pl.BlockSpec((lb, 1), lambda p, i: (i, 0)),
            pl.BlockSpec((1, h1), lambda p, i: (0, 0)),
            pl.BlockSpec((h1, h2), lambda p, i: (0, 0)),
            pl.BlockSpec((1, h2), lambda p, i: (0, 0)),
        ],
        out_specs=pl.BlockSpec((lb, h2), lambda p, i: (i, 0)),
        out_shape=jax.ShapeDtypeStruct((n, h2), jnp.float32),
        scratch_shapes=[
            pltpu.VMEM((n, h1), jnp.bfloat16),
            pltpu.VMEM((n, h2), jnp.bfloat16),
        ],
    )(adjb, z, deg, deg, b1r, W2, b2r)

    return x2
